# baseline (speedup 1.0000x reference)
import functools

import numpy as np
import jax
import jax.numpy as jnp
from jax import lax
from jax.experimental import pallas as pl
from jax.experimental.pallas import tpu as pltpu
from jax.experimental.pallas import tpu_sc as plsc

_BATCH = 4096
_D = 128
_NSP = 26
_VOCAB = 100000
_NIDX = _BATCH * _NSP
_WIN = 128
_NWIN = _NIDX // _WIN
_F = _NSP + 1
_FP = 32
_R = 256


def _sc_gather(table, idx32):
    n = idx32.shape[0]
    nwin = n // _WIN
    mesh = plsc.VectorSubcoreMesh(core_axis_name="core",
                                  subcore_axis_name="subcore")
    idx2 = idx32.reshape(1, n)

    @functools.partial(
        pl.kernel,
        out_type=jax.ShapeDtypeStruct((n, _D), jnp.float32),
        mesh=mesh)
    def gather_kernel(x_hbm, i_hbm, o_hbm):
        def body(i_vmem, o_vmem):
            pltpu.sync_copy(x_hbm.at[i_vmem.at[0]], o_vmem)

        pltpu.emit_pipeline(
            body,
            grid=(nwin,),
            in_specs=[pl.BlockSpec((1, _WIN), index_map=lambda i: (0, i))],
            out_specs=[pl.BlockSpec((_WIN, _D), index_map=lambda i: (i, 0))],
            core_axis_name=("core", "subcore"),
            dimension_semantics=(pltpu.PARALLEL,),
        )(i_hbm, o_hbm)

    return gather_kernel(table, idx2)


def _dense_body(x_ref, emb_ref, bw0, bb0, bw1, bb1, bw2, bb2,
                w0a, w0s3, tb0, tw1, tb1, tw2, tb2, tw3, tb3, tw4, tb4,
                out_ref):
    f32 = jnp.float32
    bf = jnp.bfloat16
    h = x_ref[...].astype(bf)
    h = jnp.maximum(jnp.dot(h, bw0[...], preferred_element_type=f32) + bb0[...], 0.0)
    h = jnp.maximum(jnp.dot(h.astype(bf), bw1[...], preferred_element_type=f32) + bb1[...], 0.0)
    bot = jnp.maximum(jnp.dot(h.astype(bf), bw2[...], preferred_element_type=f32) + bb2[...], 0.0)
    botb = bot.astype(bf)
    emb32 = emb_ref[...].astype(bf).reshape(_R, _FP, _D)
    fiota = lax.broadcasted_iota(jnp.int32, (1, _FP, 1), 1)
    stack = jnp.where(fiota == _NSP, botb.reshape(_R, 1, _D),
                      jnp.where(fiota < _NSP, emb32,
                                jnp.zeros((), bf)))
    xact = lax.dot_general(stack, stack, (((2,), (2,)), ((0,), (0,))),
                           preferred_element_type=f32)
    xflat = xact.astype(bf).reshape(_R, _FP * _FP)
    acc = (jnp.dot(botb, w0a[...], preferred_element_type=f32)
           + jnp.dot(xflat, w0s3[...], preferred_element_type=f32) + tb0[...])
    h = jnp.maximum(acc, 0.0)
    h = jnp.maximum(jnp.dot(h.astype(bf), tw1[...], preferred_element_type=f32) + tb1[...], 0.0)
    h = jnp.maximum(jnp.dot(h.astype(bf), tw2[...], preferred_element_type=f32) + tb2[...], 0.0)
    h = jnp.maximum(jnp.dot(h.astype(bf), tw3[...], preferred_element_type=f32) + tb3[...], 0.0)
    out_ref[...] = jnp.dot(h.astype(bf), tw4[...], preferred_element_type=f32) + tb4[...]


def _dense(x, emb2, *ws):
    nb = x.shape[0]
    specs = [pl.BlockSpec((_R, 13), lambda i: (i, 0)),
             pl.BlockSpec((_R * _FP, _D), lambda i: (i, 0))]
    for w in ws:
        specs.append(pl.BlockSpec(w.shape, lambda i, n=w.ndim: (0,) * n))
    return pl.pallas_call(
        _dense_body,
        grid=(nb // _R,),
        in_specs=specs,
        out_specs=pl.BlockSpec((_R, 1), lambda i: (i, 0)),
        out_shape=jax.ShapeDtypeStruct((nb, 1), jnp.float32),
    )(x, emb2, *ws)


_NPAIR = _F * (_F + 1) // 2
_PERM = np.array([i + 1 for i in range(_NSP)] + [0], np.int32)
_PAIR_ID = np.full((_F, _F), -1, np.int32)
_iu0, _iu1 = np.triu_indices(_F)
_PAIR_ID[_iu0, _iu1] = np.arange(_NPAIR)
_PAIR_ID[_iu1, _iu0] = _PAIR_ID[_iu0, _iu1]
_TRIU_MAP = np.full((_FP, _FP), _NPAIR, np.int32)
for _u in range(_F):
    for _v in range(_u, _F):
        _TRIU_MAP[_u, _v] = _PAIR_ID[_PERM[_u], _PERM[_v]]


def kernel(bot_mlp_input, cat_features, embedding_table,
           bot_W0, bot_b0, bot_W1, bot_b1, bot_W2, bot_b2,
           top_W0, top_b0, top_W1, top_b1, top_W2, top_b2,
           top_W3, top_b3, top_W4, top_b4):
    offs = jnp.arange(_NSP, dtype=jnp.int32) * _VOCAB
    idx32 = jnp.pad(cat_features.astype(jnp.int32) + offs[None, :],
                    ((0, 0), (0, _FP - _NSP))).reshape(-1)

    n_out = top_W0.shape[1]
    bf = jnp.bfloat16
    w0a = top_W0[:_D].astype(bf)
    w0pad = jnp.concatenate(
        [top_W0[_D:], jnp.zeros((1, n_out), jnp.float32)], axis=0).astype(bf)
    w0s3 = w0pad[jnp.asarray(_TRIU_MAP.reshape(-1))]

    row = lambda b: b.reshape(1, -1)
    ws = (bot_W0.astype(bf), row(bot_b0), bot_W1.astype(bf),
          row(bot_b1), bot_W2.astype(bf), row(bot_b2),
          w0a, w0s3, row(top_b0), top_W1.astype(bf), row(top_b1),
          top_W2.astype(bf), row(top_b2), top_W3.astype(bf),
          row(top_b3), top_W4.astype(bf), row(top_b4))

    nsl = 2
    sl = _BATCH // nsl
    embs = []
    for k in range(nsl):
        idx_k = lax.dynamic_slice_in_dim(idx32, k * sl * _FP, sl * _FP)
        embs.append(_sc_gather(embedding_table, idx_k))
    outs = []
    for k in range(nsl):
        x_k = lax.dynamic_slice_in_dim(bot_mlp_input, k * sl, sl)
        outs.append(_dense(x_k, embs[k], *ws))
    return jnp.concatenate(outs, axis=0)


# speedup vs baseline: 8.4173x; 8.4173x over previous
import functools

import numpy as np
import jax
import jax.numpy as jnp
from jax import lax
from jax.experimental import pallas as pl
from jax.experimental.pallas import tpu as pltpu
from jax.experimental.pallas import tpu_sc as plsc

_BATCH = 4096
_D = 128
_NSP = 26
_VOCAB = 100000
_NIDX = _BATCH * _NSP
_WIN = 128
_NWIN = _NIDX // _WIN
_F = _NSP + 1
_FP = 32
_R = 256


def _sc_gather(table, idx32):
    n = idx32.shape[0]
    nwin = n // _WIN
    mesh = plsc.VectorSubcoreMesh(core_axis_name="core",
                                  subcore_axis_name="subcore")
    idx2 = idx32.reshape(1, n)

    @functools.partial(
        pl.kernel,
        out_type=jax.ShapeDtypeStruct((n, _D), jnp.float32),
        mesh=mesh)
    def gather_kernel(x_hbm, i_hbm, o_hbm):
        def body(i_vmem, o_vmem):
            pltpu.sync_copy(x_hbm.at[i_vmem.at[0]], o_vmem)

        pltpu.emit_pipeline(
            body,
            grid=(nwin,),
            in_specs=[pl.BlockSpec((1, _WIN), index_map=lambda i: (0, i))],
            out_specs=[pl.BlockSpec((_WIN, _D), index_map=lambda i: (i, 0))],
            core_axis_name=("core", "subcore"),
            dimension_semantics=(pltpu.PARALLEL,),
        )(i_hbm, o_hbm)

    return gather_kernel(table, idx2)


def _dense_body(x_ref, emb_ref, bw0, bb0, bw1, bb1, bw2, bb2,
                w0a, w0s3, tb0, tw1, tb1, tw2, tb2, tw3, tb3, tw4, tb4,
                out_ref):
    f32 = jnp.float32
    bf = jnp.bfloat16
    h = x_ref[...].astype(bf)
    h = jnp.maximum(jnp.dot(h, bw0[...], preferred_element_type=f32) + bb0[...], 0.0)
    h = jnp.maximum(jnp.dot(h.astype(bf), bw1[...], preferred_element_type=f32) + bb1[...], 0.0)
    bot = jnp.maximum(jnp.dot(h.astype(bf), bw2[...], preferred_element_type=f32) + bb2[...], 0.0)
    botb = bot.astype(bf)
    emb32 = emb_ref[...].astype(bf).reshape(_R, _FP, _D)
    fiota = lax.broadcasted_iota(jnp.int32, (1, _FP, 1), 1)
    stack = jnp.where(fiota == _NSP, botb.reshape(_R, 1, _D),
                      jnp.where(fiota < _NSP, emb32,
                                jnp.zeros((), bf)))
    xact = lax.dot_general(stack, stack, (((2,), (2,)), ((0,), (0,))),
                           preferred_element_type=f32)
    xflat = xact.astype(bf).reshape(_R, _FP * _FP)
    acc = (jnp.dot(botb, w0a[...], preferred_element_type=f32)
           + jnp.dot(xflat, w0s3[...], preferred_element_type=f32) + tb0[...])
    h = jnp.maximum(acc, 0.0)
    h = jnp.maximum(jnp.dot(h.astype(bf), tw1[...], preferred_element_type=f32) + tb1[...], 0.0)
    h = jnp.maximum(jnp.dot(h.astype(bf), tw2[...], preferred_element_type=f32) + tb2[...], 0.0)
    h = jnp.maximum(jnp.dot(h.astype(bf), tw3[...], preferred_element_type=f32) + tb3[...], 0.0)
    out_ref[...] = jnp.dot(h.astype(bf), tw4[...], preferred_element_type=f32) + tb4[...]


def _dense(x, emb2, *ws):
    nb = x.shape[0]
    specs = [pl.BlockSpec((_R, 13), lambda i: (i, 0)),
             pl.BlockSpec((_R * _FP, _D), lambda i: (i, 0))]
    for w in ws:
        specs.append(pl.BlockSpec(w.shape, lambda i, n=w.ndim: (0,) * n))
    return pl.pallas_call(
        _dense_body,
        grid=(nb // _R,),
        in_specs=specs,
        out_specs=pl.BlockSpec((_R, 1), lambda i: (i, 0)),
        out_shape=jax.ShapeDtypeStruct((nb, 1), jnp.float32),
    )(x, emb2, *ws)


_NPAIR = _F * (_F + 1) // 2
_PERM = np.array([i + 1 for i in range(_NSP)] + [0], np.int32)
_PAIR_ID = np.full((_F, _F), -1, np.int32)
_iu0, _iu1 = np.triu_indices(_F)
_PAIR_ID[_iu0, _iu1] = np.arange(_NPAIR)
_PAIR_ID[_iu1, _iu0] = _PAIR_ID[_iu0, _iu1]
_TRIU_MAP = np.full((_FP, _FP), _NPAIR, np.int32)
for _u in range(_F):
    for _v in range(_u, _F):
        _TRIU_MAP[_u, _v] = _PAIR_ID[_PERM[_u], _PERM[_v]]


def kernel(bot_mlp_input, cat_features, embedding_table,
           bot_W0, bot_b0, bot_W1, bot_b1, bot_W2, bot_b2,
           top_W0, top_b0, top_W1, top_b1, top_W2, top_b2,
           top_W3, top_b3, top_W4, top_b4):
    offs = jnp.arange(_NSP, dtype=jnp.int32) * _VOCAB
    idxm = cat_features.astype(jnp.int32) + offs[None, :]
    idx32 = jnp.concatenate([idxm, idxm[:, :_FP - _NSP]], axis=1).reshape(-1)

    n_out = top_W0.shape[1]
    bf = jnp.bfloat16
    w0a = top_W0[:_D].astype(bf)
    w0pad = jnp.concatenate(
        [top_W0[_D:], jnp.zeros((1, n_out), jnp.float32)], axis=0).astype(bf)
    w0s3 = w0pad[jnp.asarray(_TRIU_MAP.reshape(-1))]

    row = lambda b: b.reshape(1, -1)
    ws = (bot_W0.astype(bf), row(bot_b0), bot_W1.astype(bf),
          row(bot_b1), bot_W2.astype(bf), row(bot_b2),
          w0a, w0s3, row(top_b0), top_W1.astype(bf), row(top_b1),
          top_W2.astype(bf), row(top_b2), top_W3.astype(bf),
          row(top_b3), top_W4.astype(bf), row(top_b4))

    nsl = 2
    sl = _BATCH // nsl
    embs = []
    for k in range(nsl):
        idx_k = lax.dynamic_slice_in_dim(idx32, k * sl * _FP, sl * _FP)
        embs.append(_sc_gather(embedding_table, idx_k))
    outs = []
    for k in range(nsl):
        x_k = lax.dynamic_slice_in_dim(bot_mlp_input, k * sl, sl)
        outs.append(_dense(x_k, embs[k], *ws))
    return jnp.concatenate(outs, axis=0)


# speedup vs baseline: 8.9763x; 1.0664x over previous
import functools

import numpy as np
import jax
import jax.numpy as jnp
from jax import lax
from jax.experimental import pallas as pl
from jax.experimental.pallas import tpu as pltpu
from jax.experimental.pallas import tpu_sc as plsc

_BATCH = 4096
_D = 128
_NSP = 26
_VOCAB = 100000
_NIDX = _BATCH * _NSP
_WIN = 128
_NWIN = _NIDX // _WIN
_F = _NSP + 1
_FP = 32
_R = 256


def _sc_gather(table, idx):
    n = idx.shape[0]
    nsamp = n // _NSP
    nwin = nsamp // 4
    mesh = plsc.VectorSubcoreMesh(core_axis_name="core",
                                  subcore_axis_name="subcore")
    idx2 = jnp.pad(idx.reshape(nwin, 4 * _NSP),
                   ((0, 0), (0, _WIN - 4 * _NSP)))

    @functools.partial(
        pl.kernel,
        out_type=jax.ShapeDtypeStruct((nsamp * _FP, _D), jnp.float32),
        mesh=mesh,
        scratch_types=[pltpu.SemaphoreType.DMA])
    def gather_kernel(x_hbm, i_hbm, o_hbm, sem):
        def body(i_vmem, o_vmem):
            copies = [
                pltpu.async_copy(
                    x_hbm.at[i_vmem.at[0, pl.ds(s * _NSP, _NSP)]],
                    o_vmem.at[pl.ds(s * _FP, _NSP)], sem)
                for s in range(4)
            ]
            for c in copies:
                c.wait()

        pltpu.emit_pipeline(
            body,
            grid=(nwin,),
            in_specs=[pl.BlockSpec((1, _WIN), index_map=lambda i: (i, 0))],
            out_specs=[pl.BlockSpec((4 * _FP, _D), index_map=lambda i: (i, 0))],
            core_axis_name=("core", "subcore"),
            dimension_semantics=(pltpu.PARALLEL,),
        )(i_hbm, o_hbm)

    return gather_kernel(table, idx2)


def _dense_body(x_ref, emb_ref, bw0, bb0, bw1, bb1, bw2, bb2,
                w0a, w0s3, tb0, tw1, tb1, tw2, tb2, tw3, tb3, tw4, tb4,
                out_ref):
    f32 = jnp.float32
    bf = jnp.bfloat16
    h = x_ref[...].astype(bf)
    h = jnp.maximum(jnp.dot(h, bw0[...], preferred_element_type=f32) + bb0[...], 0.0)
    h = jnp.maximum(jnp.dot(h.astype(bf), bw1[...], preferred_element_type=f32) + bb1[...], 0.0)
    bot = jnp.maximum(jnp.dot(h.astype(bf), bw2[...], preferred_element_type=f32) + bb2[...], 0.0)
    botb = bot.astype(bf)
    emb32 = emb_ref[...].astype(bf).reshape(_R, _FP, _D)
    fiota = lax.broadcasted_iota(jnp.int32, (1, _FP, 1), 1)
    stack = jnp.where(fiota == _NSP, botb.reshape(_R, 1, _D),
                      jnp.where(fiota < _NSP, emb32,
                                jnp.zeros((), bf)))
    xact = lax.dot_general(stack, stack, (((2,), (2,)), ((0,), (0,))),
                           preferred_element_type=f32)
    xflat = xact.astype(bf).reshape(_R, _FP * _FP)
    acc = (jnp.dot(botb, w0a[...], preferred_element_type=f32)
           + jnp.dot(xflat, w0s3[...], preferred_element_type=f32) + tb0[...])
    h = jnp.maximum(acc, 0.0)
    h = jnp.maximum(jnp.dot(h.astype(bf), tw1[...], preferred_element_type=f32) + tb1[...], 0.0)
    h = jnp.maximum(jnp.dot(h.astype(bf), tw2[...], preferred_element_type=f32) + tb2[...], 0.0)
    h = jnp.maximum(jnp.dot(h.astype(bf), tw3[...], preferred_element_type=f32) + tb3[...], 0.0)
    out_ref[...] = jnp.dot(h.astype(bf), tw4[...], preferred_element_type=f32) + tb4[...]


def _dense(x, emb2, *ws):
    nb = x.shape[0]
    specs = [pl.BlockSpec((_R, 13), lambda i: (i, 0)),
             pl.BlockSpec((_R * _FP, _D), lambda i: (i, 0))]
    for w in ws:
        specs.append(pl.BlockSpec(w.shape, lambda i, n=w.ndim: (0,) * n))
    return pl.pallas_call(
        _dense_body,
        grid=(nb // _R,),
        in_specs=specs,
        out_specs=pl.BlockSpec((_R, 1), lambda i: (i, 0)),
        out_shape=jax.ShapeDtypeStruct((nb, 1), jnp.float32),
    )(x, emb2, *ws)


_NPAIR = _F * (_F + 1) // 2
_PERM = np.array([i + 1 for i in range(_NSP)] + [0], np.int32)
_PAIR_ID = np.full((_F, _F), -1, np.int32)
_iu0, _iu1 = np.triu_indices(_F)
_PAIR_ID[_iu0, _iu1] = np.arange(_NPAIR)
_PAIR_ID[_iu1, _iu0] = _PAIR_ID[_iu0, _iu1]
_TRIU_MAP = np.full((_FP, _FP), _NPAIR, np.int32)
for _u in range(_F):
    for _v in range(_u, _F):
        _TRIU_MAP[_u, _v] = _PAIR_ID[_PERM[_u], _PERM[_v]]


def kernel(bot_mlp_input, cat_features, embedding_table,
           bot_W0, bot_b0, bot_W1, bot_b1, bot_W2, bot_b2,
           top_W0, top_b0, top_W1, top_b1, top_W2, top_b2,
           top_W3, top_b3, top_W4, top_b4):
    offs = jnp.arange(_NSP, dtype=jnp.int32) * _VOCAB
    idx = (cat_features.astype(jnp.int32) + offs[None, :]).reshape(-1)

    n_out = top_W0.shape[1]
    bf = jnp.bfloat16
    w0a = top_W0[:_D].astype(bf)
    w0pad = jnp.concatenate(
        [top_W0[_D:], jnp.zeros((1, n_out), jnp.float32)], axis=0).astype(bf)
    w0s3 = w0pad[jnp.asarray(_TRIU_MAP.reshape(-1))]

    row = lambda b: b.reshape(1, -1)
    ws = (bot_W0.astype(bf), row(bot_b0), bot_W1.astype(bf),
          row(bot_b1), bot_W2.astype(bf), row(bot_b2),
          w0a, w0s3, row(top_b0), top_W1.astype(bf), row(top_b1),
          top_W2.astype(bf), row(top_b2), top_W3.astype(bf),
          row(top_b3), top_W4.astype(bf), row(top_b4))

    nsl = 2
    sl = _BATCH // nsl
    embs = []
    for k in range(nsl):
        idx_k = lax.dynamic_slice_in_dim(idx, k * sl * _NSP, sl * _NSP)
        embs.append(_sc_gather(embedding_table, idx_k))
    outs = []
    for k in range(nsl):
        x_k = lax.dynamic_slice_in_dim(bot_mlp_input, k * sl, sl)
        outs.append(_dense(x_k, embs[k], *ws))
    return jnp.concatenate(outs, axis=0)


# speedup vs baseline: 9.2332x; 1.0286x over previous
import functools

import numpy as np
import jax
import jax.numpy as jnp
from jax import lax
from jax.experimental import pallas as pl
from jax.experimental.pallas import tpu as pltpu
from jax.experimental.pallas import tpu_sc as plsc

_BATCH = 4096
_D = 128
_NSP = 26
_VOCAB = 100000
_NIDX = _BATCH * _NSP
_WIN = 128
_NWIN = _NIDX // _WIN
_F = _NSP + 1
_FP = 32
_R = 512


def _sc_gather(table, idx):
    n = idx.shape[0]
    nsamp = n // _NSP
    nwin = nsamp // 4
    mesh = plsc.VectorSubcoreMesh(core_axis_name="core",
                                  subcore_axis_name="subcore")
    idx2 = jnp.pad(idx.reshape(nwin, 4 * _NSP),
                   ((0, 0), (0, _WIN - 4 * _NSP)))

    @functools.partial(
        pl.kernel,
        out_type=jax.ShapeDtypeStruct((nsamp * _FP, _D), jnp.float32),
        mesh=mesh,
        scratch_types=[pltpu.SemaphoreType.DMA])
    def gather_kernel(x_hbm, i_hbm, o_hbm, sem):
        def body(i_vmem, o_vmem):
            copies = [
                pltpu.async_copy(
                    x_hbm.at[i_vmem.at[0, pl.ds(s * _NSP, _NSP)]],
                    o_vmem.at[pl.ds(s * _FP, _NSP)], sem)
                for s in range(4)
            ]
            for c in copies:
                c.wait()

        pltpu.emit_pipeline(
            body,
            grid=(nwin,),
            in_specs=[pl.BlockSpec((1, _WIN), index_map=lambda i: (i, 0))],
            out_specs=[pl.BlockSpec((4 * _FP, _D), index_map=lambda i: (i, 0))],
            core_axis_name=("core", "subcore"),
            dimension_semantics=(pltpu.PARALLEL,),
        )(i_hbm, o_hbm)

    return gather_kernel(table, idx2)


def _dense_body(x_ref, emb_ref, bw0, bb0, bw1, bb1, bw2, bb2,
                w0a, w0s3, tb0, tw1, tb1, tw2, tb2, tw3, tb3, tw4, tb4,
                out_ref):
    f32 = jnp.float32
    bf = jnp.bfloat16
    h = x_ref[...].astype(bf)
    h = jnp.maximum(jnp.dot(h, bw0[...], preferred_element_type=f32) + bb0[...], 0.0)
    h = jnp.maximum(jnp.dot(h.astype(bf), bw1[...], preferred_element_type=f32) + bb1[...], 0.0)
    bot = jnp.maximum(jnp.dot(h.astype(bf), bw2[...], preferred_element_type=f32) + bb2[...], 0.0)
    botb = bot.astype(bf)
    emb32 = emb_ref[...].astype(bf).reshape(_R, _FP, _D)
    fiota = lax.broadcasted_iota(jnp.int32, (1, _FP, 1), 1)
    stack = jnp.where(fiota == _NSP, botb.reshape(_R, 1, _D),
                      jnp.where(fiota < _NSP, emb32,
                                jnp.zeros((), bf)))
    xact = lax.dot_general(stack, stack, (((2,), (2,)), ((0,), (0,))),
                           preferred_element_type=f32)
    xflat = xact.astype(bf).reshape(_R, _FP * _FP)
    acc = (jnp.dot(botb, w0a[...], preferred_element_type=f32)
           + jnp.dot(xflat, w0s3[...], preferred_element_type=f32) + tb0[...])
    h = jnp.maximum(acc, 0.0)
    h = jnp.maximum(jnp.dot(h.astype(bf), tw1[...], preferred_element_type=f32) + tb1[...], 0.0)
    h = jnp.maximum(jnp.dot(h.astype(bf), tw2[...], preferred_element_type=f32) + tb2[...], 0.0)
    h = jnp.maximum(jnp.dot(h.astype(bf), tw3[...], preferred_element_type=f32) + tb3[...], 0.0)
    out_ref[...] = jnp.dot(h.astype(bf), tw4[...], preferred_element_type=f32) + tb4[...]


def _dense(x, emb2, *ws):
    nb = x.shape[0]
    specs = [pl.BlockSpec((_R, 13), lambda i: (i, 0)),
             pl.BlockSpec((_R * _FP, _D), lambda i: (i, 0))]
    for w in ws:
        specs.append(pl.BlockSpec(w.shape, lambda i, n=w.ndim: (0,) * n))
    return pl.pallas_call(
        _dense_body,
        grid=(nb // _R,),
        in_specs=specs,
        out_specs=pl.BlockSpec((_R, 1), lambda i: (i, 0)),
        out_shape=jax.ShapeDtypeStruct((nb, 1), jnp.float32),
    )(x, emb2, *ws)


_NPAIR = _F * (_F + 1) // 2
_PERM = np.array([i + 1 for i in range(_NSP)] + [0], np.int32)
_PAIR_ID = np.full((_F, _F), -1, np.int32)
_iu0, _iu1 = np.triu_indices(_F)
_PAIR_ID[_iu0, _iu1] = np.arange(_NPAIR)
_PAIR_ID[_iu1, _iu0] = _PAIR_ID[_iu0, _iu1]
_TRIU_MAP = np.full((_FP, _FP), _NPAIR, np.int32)
for _u in range(_F):
    for _v in range(_u, _F):
        _TRIU_MAP[_u, _v] = _PAIR_ID[_PERM[_u], _PERM[_v]]


def kernel(bot_mlp_input, cat_features, embedding_table,
           bot_W0, bot_b0, bot_W1, bot_b1, bot_W2, bot_b2,
           top_W0, top_b0, top_W1, top_b1, top_W2, top_b2,
           top_W3, top_b3, top_W4, top_b4):
    offs = jnp.arange(_NSP, dtype=jnp.int32) * _VOCAB
    idx = (cat_features.astype(jnp.int32) + offs[None, :]).reshape(-1)

    n_out = top_W0.shape[1]
    bf = jnp.bfloat16
    w0a = top_W0[:_D].astype(bf)
    w0pad = jnp.concatenate(
        [top_W0[_D:], jnp.zeros((1, n_out), jnp.float32)], axis=0).astype(bf)
    w0s3 = w0pad[jnp.asarray(_TRIU_MAP.reshape(-1))]

    row = lambda b: b.reshape(1, -1)
    ws = (bot_W0.astype(bf), row(bot_b0), bot_W1.astype(bf),
          row(bot_b1), bot_W2.astype(bf), row(bot_b2),
          w0a, w0s3, row(top_b0), top_W1.astype(bf), row(top_b1),
          top_W2.astype(bf), row(top_b2), top_W3.astype(bf),
          row(top_b3), top_W4.astype(bf), row(top_b4))

    nsl = 2
    sl = _BATCH // nsl
    embs = []
    for k in range(nsl):
        idx_k = lax.dynamic_slice_in_dim(idx, k * sl * _NSP, sl * _NSP)
        embs.append(_sc_gather(embedding_table, idx_k))
    outs = []
    for k in range(nsl):
        x_k = lax.dynamic_slice_in_dim(bot_mlp_input, k * sl, sl)
        outs.append(_dense(x_k, embs[k], *ws))
    return jnp.concatenate(outs, axis=0)
